# baseline (speedup 1.0000x reference)
import numpy as np
import jax
import jax.numpy as jnp
from jax import lax
from jax.experimental import pallas as pl
from jax.experimental.pallas import tpu as pltpu
from jax.experimental.pallas import tpu_sc as plsc

_N_LABELS = 2048
_MAX_DEPTH = 4
_BATCH = 4096
_N_ROWS = 88
_N_WORDS = _N_LABELS // 4
_NW = 32
_N_REP = 16
_B_PER_W = _BATCH // _NW
_L = 16


def _build_onehot():
    a = np.zeros((_N_ROWS + 8, 16), np.float32)
    r = 1
    for d in range(1, _MAX_DEPTH):
        for g in range(4 ** d):
            a[r, d - 1] = 1.0
            a[r, 4] = -float(g)
            r += 1
    for d, off in ((1, 1.0), (2, 5.0), (3, 21.0)):
        a[_N_ROWS + d, d - 1] = 1.0
        a[_N_ROWS + d, 4] = off
    return a


_A_CONST = _build_onehot()


def _tables_body(pool_ref, a_ref, t_ref, r_ref):
    pf = pool_ref[...].astype(jnp.float32)
    pe = jnp.concatenate([
        pf,
        jnp.ones((_N_LABELS, 1), jnp.float32),
        jnp.zeros((_N_LABELS, 11), jnp.float32),
    ], axis=1)
    a = a_ref[0:_N_ROWS, :]
    t = jnp.zeros((_N_ROWS, _N_WORDS), jnp.int32)
    for k in range(4):
        blk = pe[k * _N_WORDS:(k + 1) * _N_WORDS, :]
        s = lax.dot_general(a, blk, (((1,), (1,)), ((), ())),
                            preferred_element_type=jnp.float32)
        t = t | ((s == 0.0).astype(jnp.int32) << (8 * k))
    for i in range(_N_REP):
        t_ref[pl.ds(i * _N_ROWS, _N_ROWS), :] = t
    e = a_ref[_N_ROWS:_N_ROWS + 4, :]
    rr = lax.dot_general(e, pe, (((1,), (1,)), ((), ())),
                         preferred_element_type=jnp.float32)
    r_ref[...] = jnp.clip(rr, 0.0, 84.0).astype(jnp.int32)


_build_tables = pl.pallas_call(
    _tables_body,
    out_shape=(
        jax.ShapeDtypeStruct((_N_REP * _N_ROWS, _N_WORDS), jnp.int32),
        jax.ShapeDtypeStruct((_MAX_DEPTH, _N_LABELS), jnp.int32),
    ),
)

_N_CHUNK = 4
_ROWS_PER_CHUNK = _B_PER_W // _N_CHUNK


def _sc_body(t_hbm, r_hbm, yl_hbm, d_hbm, out_hbm,
             rof_v, yl_v, d_v, ridx_v, rows_v,
             gs0, gs1, gs2, gs3, wsem, rsem, ysem):
    wid = lax.axis_index("s") * 2 + lax.axis_index("c")
    base = wid * _B_PER_W
    with jax.named_scope("inputs"):
        rload = pltpu.async_copy(r_hbm, rof_v, rsem)
        yload = pltpu.async_copy(yl_hbm.at[pl.ds(base, _B_PER_W)], yl_v,
                                 ysem)
        pltpu.sync_copy(d_hbm.at[pl.ds(base, _B_PER_W)], d_v)
        yload.wait()
        rload.wait()

    with jax.named_scope("rowids"):
        toff = (wid >> 1) * _N_ROWS
        for k in range(_B_PER_W // _L):
            yl = yl_v[pl.ds(k * _L, _L)]
            dv = d_v[pl.ds(k * _L, _L)]
            ridx_v[pl.ds(k * _L, _L)] = toff + plsc.load_gather(
                rof_v, [dv, yl])

    gsems = (gs0, gs1, gs2, gs3)
    with jax.named_scope("gather_start"):
        gathers = [
            pltpu.async_copy(
                t_hbm.at[ridx_v.at[pl.ds(k * _ROWS_PER_CHUNK,
                                         _ROWS_PER_CHUNK)]],
                rows_v.at[k], gsems[k])
            for k in range(_N_CHUNK)
        ]
    writes = []
    for k in range(_N_CHUNK):
        with jax.named_scope(f"gwait{k}"):
            gathers[k].wait()
        with jax.named_scope(f"wstart{k}"):
            writes.append(pltpu.async_copy(
                rows_v.at[k],
                out_hbm.at[pl.ds(base + k * _ROWS_PER_CHUNK,
                                 _ROWS_PER_CHUNK)],
                wsem))
    with jax.named_scope("wwait"):
        for w in writes:
            w.wait()


_SC_GATHER = None


def _sc_gather():
    global _SC_GATHER
    if _SC_GATHER is None:
        _SC_GATHER = pl.kernel(
            _sc_body,
            out_type=jax.ShapeDtypeStruct((_BATCH, _N_WORDS), jnp.int32),
            mesh=plsc.VectorSubcoreMesh(core_axis_name="c",
                                        subcore_axis_name="s"),
            scratch_types=[
                pltpu.VMEM((_MAX_DEPTH, _N_LABELS), jnp.int32),
                pltpu.VMEM((_B_PER_W,), jnp.int32),
                pltpu.VMEM((_B_PER_W,), jnp.int32),
                pltpu.VMEM((_B_PER_W,), jnp.int32),
                pltpu.VMEM((_N_CHUNK, _ROWS_PER_CHUNK, _N_WORDS), jnp.int32),
                pltpu.SemaphoreType.DMA,
                pltpu.SemaphoreType.DMA,
                pltpu.SemaphoreType.DMA,
                pltpu.SemaphoreType.DMA,
                pltpu.SemaphoreType.DMA,
                pltpu.SemaphoreType.DMA,
                pltpu.SemaphoreType.DMA,
            ],
            compiler_params=pltpu.CompilerParams(needs_layout_passes=False),
        )
    return _SC_GATHER


def kernel(y, depths, label_pool):
    yl = y[:, -1]
    dd = depths[:, 0]
    t, r = _build_tables(label_pool, jnp.asarray(_A_CONST))
    words = _sc_gather()(t, r, yl, dd)
    return jnp.concatenate(
        [((words >> (8 * k)) & 1).astype(jnp.bool_) for k in range(4)],
        axis=1)


# speedup vs baseline: 1.0521x; 1.0521x over previous
import numpy as np
import jax
import jax.numpy as jnp
from jax import lax
from jax.experimental import pallas as pl
from jax.experimental.pallas import tpu as pltpu
from jax.experimental.pallas import tpu_sc as plsc

_N_LABELS = 2048
_MAX_DEPTH = 4
_BATCH = 4096
_N_ROWS = 88
_N_WORDS = _N_LABELS // 4
_NW = 32
_N_REP = 32
_B_PER_W = _BATCH // _NW
_L = 16


def _build_onehot():
    a = np.zeros((_N_ROWS + 8, 16), np.float32)
    r = 1
    for d in range(1, _MAX_DEPTH):
        for g in range(4 ** d):
            a[r, d - 1] = 1.0
            a[r, 4] = -float(g)
            r += 1
    for d, off in ((1, 1.0), (2, 5.0), (3, 21.0)):
        a[_N_ROWS + d, d - 1] = 1.0
        a[_N_ROWS + d, 4] = off
    return a


_A_CONST = _build_onehot()


def _tables_body(pool_ref, a_ref, t_ref, r_ref):
    pf = pool_ref[...].astype(jnp.float32)
    pe = jnp.concatenate([
        pf,
        jnp.ones((_N_LABELS, 1), jnp.float32),
        jnp.zeros((_N_LABELS, 11), jnp.float32),
    ], axis=1)
    a = a_ref[0:_N_ROWS, :]
    t = jnp.zeros((_N_ROWS, _N_WORDS), jnp.int32)
    for k in range(4):
        blk = pe[k * _N_WORDS:(k + 1) * _N_WORDS, :]
        s = lax.dot_general(a, blk, (((1,), (1,)), ((), ())),
                            preferred_element_type=jnp.float32)
        t = t | ((s == 0.0).astype(jnp.int32) << (8 * k))
    for i in range(_N_REP):
        t_ref[pl.ds(i * _N_ROWS, _N_ROWS), :] = t
    e = a_ref[_N_ROWS:_N_ROWS + 4, :]
    rr = lax.dot_general(e, pe, (((1,), (1,)), ((), ())),
                         preferred_element_type=jnp.float32)
    r_ref[...] = jnp.clip(rr, 0.0, 84.0).astype(jnp.int32)


_build_tables = pl.pallas_call(
    _tables_body,
    out_shape=(
        jax.ShapeDtypeStruct((_N_REP * _N_ROWS, _N_WORDS), jnp.int32),
        jax.ShapeDtypeStruct((_MAX_DEPTH, _N_LABELS), jnp.int32),
    ),
)

_N_CHUNK = 8
_ROWS_PER_CHUNK = _B_PER_W // _N_CHUNK


def _sc_body(t_hbm, r_hbm, yl_hbm, d_hbm, out_hbm,
             rof_v, yl_v, d_v, ridx_v, rows_v,
             gs0, gs1, gs2, gs3, gs4, gs5, gs6, gs7, wsem, rsem, ysem):
    wid = lax.axis_index("s") * 2 + lax.axis_index("c")
    base = wid * _B_PER_W
    with jax.named_scope("inputs"):
        rload = pltpu.async_copy(r_hbm, rof_v, rsem)
        yload = pltpu.async_copy(yl_hbm.at[pl.ds(base, _B_PER_W)], yl_v,
                                 ysem)
        pltpu.sync_copy(d_hbm.at[pl.ds(base, _B_PER_W)], d_v)
        yload.wait()
        rload.wait()

    with jax.named_scope("rowids"):
        toff = wid * _N_ROWS
        for k in range(_B_PER_W // _L):
            yl = yl_v[pl.ds(k * _L, _L)]
            dv = d_v[pl.ds(k * _L, _L)]
            ridx_v[pl.ds(k * _L, _L)] = toff + plsc.load_gather(
                rof_v, [dv, yl])

    gsems = (gs0, gs1, gs2, gs3, gs4, gs5, gs6, gs7)
    with jax.named_scope("gather_start"):
        gathers = [
            pltpu.async_copy(
                t_hbm.at[ridx_v.at[pl.ds(k * _ROWS_PER_CHUNK,
                                         _ROWS_PER_CHUNK)]],
                rows_v.at[k], gsems[k])
            for k in range(_N_CHUNK)
        ]
    writes = []
    for k in range(_N_CHUNK):
        with jax.named_scope(f"gwait{k}"):
            gathers[k].wait()
        with jax.named_scope(f"wstart{k}"):
            writes.append(pltpu.async_copy(
                rows_v.at[k],
                out_hbm.at[pl.ds(base + k * _ROWS_PER_CHUNK,
                                 _ROWS_PER_CHUNK)],
                wsem))
    with jax.named_scope("wwait"):
        for w in writes:
            w.wait()


_SC_GATHER = None


def _sc_gather():
    global _SC_GATHER
    if _SC_GATHER is None:
        _SC_GATHER = pl.kernel(
            _sc_body,
            out_type=jax.ShapeDtypeStruct((_BATCH, _N_WORDS), jnp.int32),
            mesh=plsc.VectorSubcoreMesh(core_axis_name="c",
                                        subcore_axis_name="s"),
            scratch_types=[
                pltpu.VMEM((_MAX_DEPTH, _N_LABELS), jnp.int32),
                pltpu.VMEM((_B_PER_W,), jnp.int32),
                pltpu.VMEM((_B_PER_W,), jnp.int32),
                pltpu.VMEM((_B_PER_W,), jnp.int32),
                pltpu.VMEM((_N_CHUNK, _ROWS_PER_CHUNK, _N_WORDS), jnp.int32),
                pltpu.SemaphoreType.DMA,
                pltpu.SemaphoreType.DMA,
                pltpu.SemaphoreType.DMA,
                pltpu.SemaphoreType.DMA,
                pltpu.SemaphoreType.DMA,
                pltpu.SemaphoreType.DMA,
                pltpu.SemaphoreType.DMA,
                pltpu.SemaphoreType.DMA,
                pltpu.SemaphoreType.DMA,
                pltpu.SemaphoreType.DMA,
                pltpu.SemaphoreType.DMA,
            ],
            compiler_params=pltpu.CompilerParams(needs_layout_passes=False),
        )
    return _SC_GATHER


def kernel(y, depths, label_pool):
    yl = y[:, -1]
    dd = depths[:, 0]
    t, r = _build_tables(label_pool, jnp.asarray(_A_CONST))
    words = _sc_gather()(t, r, yl, dd)
    return jnp.concatenate(
        [((words >> (8 * k)) & 1).astype(jnp.bool_) for k in range(4)],
        axis=1)
